# Initial kernel scaffold; baseline (speedup 1.0000x reference)
import jax
import jax.numpy as jnp
from jax.experimental import pallas as pl


def kernel(phrase, phrase_emb, pos_emb):
    raise NotImplementedError("write your pallas kernel here")


# speedup vs baseline: 1.9518x; 1.9518x over previous
import functools

import jax
import jax.numpy as jnp
from jax import lax
from jax.experimental import pallas as pl
from jax.experimental.pallas import tpu as pltpu
from jax.experimental.pallas import tpu_sc as plsc


def _phrase_embed_sc(idx2d, phrase_emb, pos2):
    NC, NS = 2, 16
    NW = NC * NS
    n_chunks, CHUNK = idx2d.shape
    _, D = phrase_emb.shape
    CPW = n_chunks // NW
    G = D // 16

    mesh = plsc.VectorSubcoreMesh(core_axis_name="c", subcore_axis_name="s")

    @functools.partial(
        pl.kernel,
        out_type=jax.ShapeDtypeStruct((n_chunks, CHUNK, D), jnp.float32),
        mesh=mesh,
        scratch_types=[
            pltpu.VMEM((CPW, CHUNK), jnp.int32),
            pltpu.VMEM((CHUNK, D), jnp.float32),
            pltpu.VMEM((CHUNK, D), jnp.float32),
            pltpu.SemaphoreType.DMA,
        ],
        compiler_params=pltpu.CompilerParams(use_tc_tiling_on_sc=False),
    )
    def k(idx_hbm, emb_hbm, pos_hbm, out_hbm, idx_v, pos_v, rows_v, gsem):
        wid = lax.axis_index("s") * NC + lax.axis_index("c")
        crow = wid * CPW
        pltpu.sync_copy(idx_hbm.at[pl.ds(crow, CPW)], idx_v)
        pltpu.sync_copy(pos_hbm, pos_v)

        @pl.loop(0, CPW)
        def _chunk(j):
            pltpu.async_copy(emb_hbm.at[idx_v.at[j]], rows_v, gsem).wait()

            @pl.loop(0, CHUNK)
            def _tok(t):
                for g in range(G):
                    sl = pl.ds(g * 16, 16)
                    plsc.addupdate(rows_v.at[t, sl], pos_v[t, sl])

            pltpu.sync_copy(rows_v, out_hbm.at[crow + j])

    return k(idx2d, phrase_emb, pos2)


def kernel(phrase, phrase_emb, pos_emb):
    B, L = phrase.shape
    _, D = phrase_emb.shape
    CHUNK = 2 * L
    idx2d = phrase.reshape(B * L // CHUNK, CHUNK)
    pos2 = jnp.concatenate([pos_emb[:L], pos_emb[:L]], axis=0)
    out = _phrase_embed_sc(idx2d, phrase_emb, pos2)
    return out.reshape(B, L, D)


# speedup vs baseline: 2.1282x; 1.0903x over previous
import functools

import jax
import jax.numpy as jnp
from jax import lax
from jax.experimental import pallas as pl
from jax.experimental.pallas import tpu as pltpu
from jax.experimental.pallas import tpu_sc as plsc


def _phrase_embed_sc(idx2d, phrase_emb, pos2):
    NC, NS = 2, 16
    NW = NC * NS
    n_chunks, CHUNK = idx2d.shape
    _, D = phrase_emb.shape
    L = CHUNK // 2
    CPW = n_chunks // NW
    G = D // 16
    NBUF = 4

    mesh = plsc.VectorSubcoreMesh(core_axis_name="c", subcore_axis_name="s")

    @functools.partial(
        pl.kernel,
        out_type=jax.ShapeDtypeStruct((n_chunks, CHUNK, D), jnp.float32),
        mesh=mesh,
        scratch_types=[
            pltpu.VMEM((CPW, CHUNK), jnp.int32),
            pltpu.VMEM((CHUNK, D), jnp.float32),
            pltpu.VMEM((NBUF, CHUNK, D), jnp.float32),
            pltpu.SemaphoreType.DMA((NBUF,)),
            pltpu.SemaphoreType.DMA((NBUF,)),
        ],
        compiler_params=pltpu.CompilerParams(use_tc_tiling_on_sc=False),
    )
    def k(idx_hbm, emb_hbm, pos_hbm, out_hbm, idx_v, pos_v, rows_v, gsem, ssem):
        wid = lax.axis_index("s") * NC + lax.axis_index("c")
        crow = wid * CPW
        pltpu.sync_copy(idx_hbm.at[pl.ds(crow, CPW)], idx_v)
        pltpu.sync_copy(pos_hbm, pos_v)

        def start_gather(j, b):
            pltpu.async_copy(emb_hbm.at[idx_v.at[j]], rows_v.at[b], gsem.at[b])

        for b in range(NBUF):
            start_gather(b, b)

        @pl.loop(0, CPW, step=NBUF)
        def _round(j0):
            for b in range(NBUF):
                j = j0 + b
                pltpu.make_async_copy(
                    emb_hbm.at[idx_v.at[j]], rows_v.at[b], gsem.at[b]
                ).wait()

                @pl.loop(0, L)
                def _tok(t):
                    for g in range(G):
                        sl = pl.ds(g * 16, 16)
                        p = pos_v[t, sl]
                        plsc.addupdate(rows_v.at[b, t, sl], p)
                        plsc.addupdate(rows_v.at[b, t + L, sl], p)

                pltpu.async_copy(rows_v.at[b], out_hbm.at[crow + j], ssem.at[b])

                pb = (b - 1) % NBUF
                jp = j - 1

                @pl.when(jp >= 0)
                def _():
                    pltpu.make_async_copy(
                        rows_v.at[pb], out_hbm.at[crow + jp], ssem.at[pb]
                    ).wait()

                    @pl.when(jp + NBUF < CPW)
                    def _():
                        start_gather(jp + NBUF, pb)

        pltpu.make_async_copy(
            rows_v.at[NBUF - 1], out_hbm.at[crow + CPW - 1], ssem.at[NBUF - 1]
        ).wait()

    return k(idx2d, phrase_emb, pos2)


def kernel(phrase, phrase_emb, pos_emb):
    B, L = phrase.shape
    _, D = phrase_emb.shape
    CHUNK = 2 * L
    idx2d = phrase.reshape(B * L // CHUNK, CHUNK)
    pos2 = jnp.concatenate([pos_emb[:L], pos_emb[:L]], axis=0)
    out = _phrase_embed_sc(idx2d, phrase_emb, pos2)
    return out.reshape(B, L, D)
